# baseline (speedup 1.0000x reference)
import jax
import jax.numpy as jnp
from jax.experimental import pallas as pl
from jax.experimental.pallas import tpu as pltpu

_N = 4000000
_S = 262144
_ROWS = 31250
_SB_ROWS = 2048
_NSB = 16
_MROWS = 256
_CLIP_LO = 1e-4
_CLIP_HI = 1.0 - 1e-4


def _masked_logs(a, g):
    rows = g * _SB_ROWS + jax.lax.broadcasted_iota(jnp.int32, (_SB_ROWS, 128), 0)
    return jnp.where(rows < _ROWS,
                     jnp.log(jnp.clip(a, _CLIP_LO, _CLIP_HI)),
                     jnp.float32(0.0))


def _scan_a_body(a_ref, bsum_ref, logsT_ref):
    logs = _masked_logs(a_ref[...], pl.program_id(0))
    for q in range(16):
        logsT_ref[:, q, :] = logs[q * 128:(q + 1) * 128, :].T

    def step(j, c):
        return c + logsT_ref[j]

    bsum_ref[...] = jax.lax.fori_loop(
        0, 128, step, jnp.zeros((16, 128), jnp.float32))


def _offs_body(bsum_ref, off1_ref, t2_ref, sq_ref, b2T_ref):
    t2_ref[:, 0:128] = bsum_ref[0:128, :].T
    t2_ref[:, 128:256] = bsum_ref[128:256, :].T

    def step2(c, carry):
        nc = carry + t2_ref[c]
        t2_ref[c] = nc
        return nc

    jax.lax.fori_loop(0, 128, step2, jnp.zeros((_MROWS,), jnp.float32))
    sq_ref[...] = jnp.zeros((128, 128), jnp.float32)
    sq_ref[0, :] = t2_ref[127, 0:128]
    sq_ref[1, :] = t2_ref[127, 128:256]
    b2T_ref[...] = sq_ref[...].T

    def step3(c, carry):
        nc = carry + b2T_ref[c]
        sq_ref[c] = nc
        return nc

    jax.lax.fori_loop(0, 128, step3, jnp.zeros((128,), jnp.float32))
    int3T = sq_ref[...].T
    r0 = int3T[0, 127]
    row0 = int3T[0:1, :]
    row1 = int3T[1:2, :] + r0
    zero1 = jnp.zeros((1, 1), jnp.float32)
    off2 = jnp.concatenate(
        [zero1, row0[:, 0:127], row0[:, 127:128], row1[:, 0:127]], axis=1)
    incl2T = t2_ref[...] + off2
    rowz = jnp.concatenate([zero1, incl2T[127:128, :-1]], axis=1)
    off1T = jnp.concatenate([rowz, incl2T[:127, :]], axis=0)
    off1_ref[0:128, :] = off1T[:, 0:128].T
    off1_ref[128:256, :] = off1T[:, 128:256].T


def _prefix_body(a_ref, off1_ref, out_ref, logsT_ref, scanT_ref):
    logs = _masked_logs(a_ref[...], pl.program_id(0))
    for q in range(16):
        logsT_ref[:, q, :] = logs[q * 128:(q + 1) * 128, :].T

    def step(j, c):
        nc = c + logsT_ref[j]
        scanT_ref[j] = nc
        return nc

    jax.lax.fori_loop(0, 128, step, jnp.zeros((16, 128), jnp.float32))
    prefT = (scanT_ref[...] + off1_ref[...][None, :, :]) - logsT_ref[...]
    for q in range(16):
        out_ref[q * 128:(q + 1) * 128, :] = prefT[:, q, :].T


_NEG = float("-inf")


def _seg_combine(m_later, f_later, m_earlier, f_earlier):
    m = jnp.where(f_later > 0, m_later, jnp.maximum(m_later, m_earlier))
    f = jnp.maximum(f_later, f_earlier)
    return m, f


def _col_scan(m, f, reverse):
    for k in (1, 2, 4, 8, 16, 32, 64):
        pad_m = jnp.full((k,) + m.shape[1:], _NEG)
        pad_f = jnp.zeros((k,) + m.shape[1:], jnp.float32)
        if not reverse:
            m_sh = jnp.concatenate([pad_m, m[:-k]], axis=0)
            f_sh = jnp.concatenate([pad_f, f[:-k]], axis=0)
        else:
            m_sh = jnp.concatenate([m[k:], pad_m], axis=0)
            f_sh = jnp.concatenate([f[k:], pad_f], axis=0)
        m, f = _seg_combine(m, f, m_sh, f_sh)
    return m, f


def _tops_scan(tm, tf, sb_m, reverse):
    axis1 = 1
    rm, rf = tm, tf
    for k in (1, 2, 4, 8, 16, 32, 64):
        pad_m = jnp.full((16, k), _NEG)
        pad_f = jnp.zeros((16, k), jnp.float32)
        if not reverse:
            m_sh = jnp.concatenate([pad_m, rm[:, :-k]], axis=axis1)
            f_sh = jnp.concatenate([pad_f, rf[:, :-k]], axis=axis1)
        else:
            m_sh = jnp.concatenate([rm[:, k:], pad_m], axis=axis1)
            f_sh = jnp.concatenate([rf[:, k:], pad_f], axis=axis1)
        rm, rf = _seg_combine(rm, rf, m_sh, f_sh)
    if not reverse:
        rt_m, rt_f = rm[:, 127:128], rf[:, 127:128]
        sm = jnp.concatenate([jnp.full((1, 1), sb_m), rt_m[:-1]], axis=0)
        sf = jnp.concatenate([jnp.zeros((1, 1), jnp.float32), rt_f[:-1]],
                             axis=0)
    else:
        rt_m, rt_f = rm[:, 0:1], rf[:, 0:1]
        sm = jnp.concatenate([rt_m[1:], jnp.full((1, 1), sb_m)], axis=0)
        sf = jnp.concatenate([rt_f[1:], jnp.zeros((1, 1), jnp.float32)],
                             axis=0)
    for k in (1, 2, 4, 8):
        pad_m = jnp.full((k, 1), _NEG)
        pad_f = jnp.zeros((k, 1), jnp.float32)
        if not reverse:
            m_sh = jnp.concatenate([pad_m, sm[:-k]], axis=0)
            f_sh = jnp.concatenate([pad_f, sf[:-k]], axis=0)
        else:
            m_sh = jnp.concatenate([sm[k:], pad_m], axis=0)
            f_sh = jnp.concatenate([sf[k:], pad_f], axis=0)
        sm, sf = _seg_combine(sm, sf, m_sh, f_sh)
    if not reverse:
        rex_m = jnp.concatenate([jnp.full((16, 1), _NEG), rm[:, :-1]],
                                axis=axis1)
        rex_f = jnp.concatenate([jnp.zeros((16, 1), jnp.float32),
                                 rf[:, :-1]], axis=axis1)
        last_m, last_f, last_c = rt_m[15, 0], rt_f[15, 0], sm[15, 0]
    else:
        rex_m = jnp.concatenate([rm[:, 1:], jnp.full((16, 1), _NEG)],
                                axis=axis1)
        rex_f = jnp.concatenate([rf[:, 1:],
                                 jnp.zeros((16, 1), jnp.float32)], axis=axis1)
        last_m, last_f, last_c = rt_m[0, 0], rt_f[0, 0], sm[0, 0]
    carry_m, _ = _seg_combine(rex_m, rex_f, sm, sf)
    new_sb = jnp.where(last_f > 0, last_m, jnp.maximum(last_m, last_c))
    return carry_m, new_sb


def _transpose16(src, dst_ref):
    for q in range(16):
        dst_ref[:, q, :] = src[q * 128:(q + 1) * 128, :].T


def _fwd_body(a_ref, off1_ref, seg_ref, segp_ref, prefT_ref, fmaxT_ref,
              logsT_ref, scanT_ref, flagT_ref, carry_ref):
    g = pl.program_id(0)
    logs = _masked_logs(a_ref[...], g)
    _transpose16(logs, logsT_ref)

    def step(j, c):
        nc = c + logsT_ref[j]
        scanT_ref[j] = nc
        return nc

    jax.lax.fori_loop(0, 128, step, jnp.zeros((16, 128), jnp.float32))
    prefT = (scanT_ref[...] + off1_ref[...][None, :, :]) - logsT_ref[...]
    prefT_ref[...] = prefT.reshape(128, 2048)

    start = (seg_ref[...] != segp_ref[...]).astype(jnp.float32)
    _transpose16(start, flagT_ref)

    @pl.when(g == 0)
    def _():
        carry_ref[0] = _NEG

    m, f = _col_scan(prefT, flagT_ref[...], reverse=False)
    sb_m = carry_ref[0]
    carry2d, new_sb = _tops_scan(m[127], f[127], sb_m, reverse=False)
    fmaxT = jnp.where(f > 0, m, jnp.maximum(m, carry2d[None, :, :]))
    fmaxT_ref[...] = fmaxT.reshape(128, 2048)
    carry_ref[0] = new_sb


def _bwd_body(prefT_ref, fmaxT_ref, seg_ref, segn_ref, a_ref, talpha_ref,
              flagT_ref, carry_ref):
    g = pl.program_id(0)
    end = (seg_ref[...] != segn_ref[...]).astype(jnp.float32)
    _transpose16(end, flagT_ref)

    @pl.when(g == 0)
    def _():
        carry_ref[0] = _NEG

    prefT = prefT_ref[...].reshape(128, 16, 128)
    m, f = _col_scan(prefT, flagT_ref[...], reverse=True)
    sb_m = carry_ref[0]
    carry2d, new_sb = _tops_scan(m[0], f[0], sb_m, reverse=True)
    bmaxT = jnp.where(f > 0, m, jnp.maximum(m, carry2d[None, :, :]))
    carry_ref[0] = new_sb

    baseT = jnp.maximum(fmaxT_ref[...].reshape(128, 16, 128), bmaxT)
    tT = jnp.exp(prefT - baseT)
    for q in range(16):
        blk = tT[:, q, :].T
        a = jnp.clip(a_ref[q * 128:(q + 1) * 128, :], _CLIP_LO, _CLIP_HI)
        talpha_ref[q * 128:(q + 1) * 128, :] = blk * (1.0 - a)


def _compute_scans(a2, seg2, segp2, segn2):
    bsum = pl.pallas_call(
        _scan_a_body,
        grid=(_NSB,),
        in_specs=[pl.BlockSpec((_SB_ROWS, 128), lambda g: (g, 0))],
        out_specs=pl.BlockSpec((16, 128), lambda g: (g, 0)),
        out_shape=jax.ShapeDtypeStruct((_MROWS, 128), jnp.float32),
        scratch_shapes=[pltpu.VMEM((128, 16, 128), jnp.float32)],
    )(a2)
    off1 = pl.pallas_call(
        _offs_body,
        out_shape=jax.ShapeDtypeStruct((_MROWS, 128), jnp.float32),
        scratch_shapes=[pltpu.VMEM((128, _MROWS), jnp.float32),
                        pltpu.VMEM((128, 128), jnp.float32),
                        pltpu.VMEM((128, 128), jnp.float32)],
    )(bsum)
    nblk = pl.BlockSpec((_SB_ROWS, 128), lambda g: (g, 0))
    tblk = pl.BlockSpec((128, 2048), lambda g: (g, 0))
    prefT, fmaxT = pl.pallas_call(
        _fwd_body,
        grid=(_NSB,),
        in_specs=[nblk,
                  pl.BlockSpec((16, 128), lambda g: (g, 0)),
                  nblk, nblk],
        out_specs=[tblk, tblk],
        out_shape=[jax.ShapeDtypeStruct((_NSB * 128, 2048), jnp.float32),
                   jax.ShapeDtypeStruct((_NSB * 128, 2048), jnp.float32)],
        scratch_shapes=[pltpu.VMEM((128, 16, 128), jnp.float32),
                        pltpu.VMEM((128, 16, 128), jnp.float32),
                        pltpu.VMEM((128, 16, 128), jnp.float32),
                        pltpu.SMEM((1,), jnp.float32)],
    )(a2, off1, seg2, segp2)
    rnblk = pl.BlockSpec((_SB_ROWS, 128), lambda g: (_NSB - 1 - g, 0))
    rtblk = pl.BlockSpec((128, 2048), lambda g: (_NSB - 1 - g, 0))
    talpha = pl.pallas_call(
        _bwd_body,
        grid=(_NSB,),
        in_specs=[rtblk, rtblk, rnblk, rnblk, rnblk],
        out_specs=rnblk,
        out_shape=jax.ShapeDtypeStruct((_ROWS, 128), jnp.float32),
        scratch_shapes=[pltpu.VMEM((128, 16, 128), jnp.float32),
                        pltpu.SMEM((1,), jnp.float32)],
    )(prefT, fmaxT, seg2, segn2, a2)
    return talpha


def kernel(anti_opacity, values, segment_ids):
    a2 = anti_opacity.reshape(_ROWS, 128)
    seg = segment_ids.astype(jnp.int32)
    segp = jnp.concatenate([jnp.full((1,), -1, jnp.int32), seg[:-1]])
    segn = jnp.concatenate([seg[1:], jnp.full((1,), -2, jnp.int32)])
    talpha = _compute_scans(a2, seg.reshape(_ROWS, 128),
                            segp.reshape(_ROWS, 128),
                            segn.reshape(_ROWS, 128)).reshape(_N)
    pix = talpha[:, None] * values
    image = jax.ops.segment_sum(pix, segment_ids, num_segments=_S,
                                indices_are_sorted=True)
    return image
